# baseline (speedup 1.0000x reference)
import functools
import math

import jax
import jax.numpy as jnp
from jax import lax
from jax.experimental import pallas as pl
from jax.experimental.pallas import tpu as pltpu
from jax.experimental.pallas import tpu_sc as plsc

_PRE = 6000
_POST = 300
_THRESH = 0.7
_NS = 16
_PAD = 6144
_SHARD = _PAD // _NS
_BLK = _SHARD // 16
_OUT_ROWS = 304
_BIG = 2 ** 30
_LOG_MAX_RATIO = math.log(1000.0 / 16.0)


def _iota16():
    return lax.broadcasted_iota(jnp.int32, (16,), 0)


def _xlane_min(v):
    dnums = lax.GatherDimensionNumbers(offset_dims=(),
                                       collapsed_slice_dims=(0,),
                                       start_index_map=(0,))
    for s in (8, 4, 2, 1):
        perm = jnp.bitwise_xor(_iota16(), s)
        shuf = lax.gather(v, perm[:, None], dnums, slice_sizes=(1,),
                          mode=lax.GatherScatterMode.PROMISE_IN_BOUNDS)
        v = jnp.minimum(v, shuf)
    return v


def _sc_nms(ax1_h, ay1_h, ax2_h, ay2_h, dx_h, dy_h, dw_h, dh_h, sc_h,
            out_h,
            ia1, ib1, ia2, ib2, idx, idy, idw, idh,
            dx1, dy1, dx2, dy2, dar,
            rep_x1, rep_y1, rep_x2, rep_y2, rep_ar, rep_sc,
            work, cand_spl, cand_buf, row_buf,
            sh_x1, sh_y1, sh_x2, sh_y2, sh_ar, sh_cand):
    cid = lax.axis_index("c")
    sid = lax.axis_index("s")
    off = sid * _SHARD

    for src, dst in ((ax1_h, ia1), (ay1_h, ib1), (ax2_h, ia2),
                     (ay2_h, ib2), (dx_h, idx), (dy_h, idy),
                     (dw_h, idw), (dh_h, idh)):
        pltpu.sync_copy(src.at[pl.ds(off, _SHARD)], dst)
    pltpu.sync_copy(sc_h.at[pl.ds(off, _SHARD)], work)
    pltpu.sync_copy(sc_h, rep_sc)

    for b in range(_BLK):
        s = b * 16
        a1 = ia1[pl.ds(s, 16)]
        b1 = ib1[pl.ds(s, 16)]
        a2 = ia2[pl.ds(s, 16)]
        b2 = ib2[pl.ds(s, 16)]
        ddx = idx[pl.ds(s, 16)]
        ddy = idy[pl.ds(s, 16)]
        ddw = idw[pl.ds(s, 16)]
        ddh = idh[pl.ds(s, 16)]
        widths = a2 - a1 + 1.0
        heights = b2 - b1 + 1.0
        ctr_x = a1 + 0.5 * widths
        ctr_y = b1 + 0.5 * heights
        ddw = jnp.minimum(ddw, _LOG_MAX_RATIO)
        ddh = jnp.minimum(ddh, _LOG_MAX_RATIO)
        pcx = ddx * widths + ctr_x
        pcy = ddy * heights + ctr_y
        pw = jnp.exp(ddw) * widths
        ph = jnp.exp(ddh) * heights
        x1 = jnp.clip(pcx - 0.5 * pw, 0.0, 1023.0)
        y1 = jnp.clip(pcy - 0.5 * ph, 0.0, 1023.0)
        x2 = jnp.clip(pcx + 0.5 * pw, 0.0, 1023.0)
        y2 = jnp.clip(pcy + 0.5 * ph, 0.0, 1023.0)
        dx1[pl.ds(s, 16)] = x1
        dy1[pl.ds(s, 16)] = y1
        dx2[pl.ds(s, 16)] = x2
        dy2[pl.ds(s, 16)] = y2
        dar[pl.ds(s, 16)] = (x2 - x1 + 1.0) * (y2 - y1 + 1.0)

    for loc, sh in ((dx1, sh_x1), (dy1, sh_y1), (dx2, sh_x2),
                    (dy2, sh_y2), (dar, sh_ar)):
        pltpu.sync_copy(loc, sh.at[pl.ds(off, _SHARD)])
    plsc.subcore_barrier()
    for sh, rep in ((sh_x1, rep_x1), (sh_y1, rep_y1), (sh_x2, rep_x2),
                    (sh_y2, rep_y2), (sh_ar, rep_ar)):
        pltpu.sync_copy(sh, rep)

    def round_body(i, carry):
        par = lax.rem(i, 2)
        m = jnp.full((16,), _BIG, jnp.int32)
        for b in range(_BLK):
            w = work[pl.ds(b * 16, 16)]
            lin = _iota16() + (off + b * 16)
            m = jnp.minimum(m, jnp.where(w > -5e8, lin, _BIG))
        cand_spl[...] = _xlane_min(m)
        pltpu.sync_copy(cand_spl,
                        sh_cand.at[pl.ds(par * (_NS * 16) + sid * 16, 16)])
        plsc.subcore_barrier()
        pltpu.sync_copy(sh_cand.at[pl.ds(par * (_NS * 16), _NS * 16)],
                        cand_buf)
        jid = cand_buf[pl.ds(0, 16)]
        for k in range(1, _NS):
            jid = jnp.minimum(jid, cand_buf[pl.ds(k * 16, 16)])
        j = jid[0]
        valid = j < _BIG

        @pl.when(valid)
        def _():
            bx1 = plsc.load_gather(rep_x1, [jid])
            by1 = plsc.load_gather(rep_y1, [jid])
            bx2 = plsc.load_gather(rep_x2, [jid])
            by2 = plsc.load_gather(rep_y2, [jid])
            bar = plsc.load_gather(rep_ar, [jid])
            bsc = plsc.load_gather(rep_sc, [jid])
            for b in range(_BLK):
                s = b * 16
                w = work[pl.ds(s, 16)]
                x1 = dx1[pl.ds(s, 16)]
                y1 = dy1[pl.ds(s, 16)]
                x2 = dx2[pl.ds(s, 16)]
                y2 = dy2[pl.ds(s, 16)]
                ar = dar[pl.ds(s, 16)]
                xx1 = jnp.maximum(bx1, x1)
                yy1 = jnp.maximum(by1, y1)
                xx2 = jnp.minimum(bx2, x2)
                yy2 = jnp.minimum(by2, y2)
                iw = jnp.maximum(0.0, xx2 - xx1 + 1.0)
                ih = jnp.maximum(0.0, yy2 - yy1 + 1.0)
                inter = iw * ih
                iou = inter / (bar + ar - inter)
                lin = _iota16() + (off + s)
                supp = jnp.logical_or(iou >= _THRESH, lin == jid)
                work[pl.ds(s, 16)] = jnp.where(supp, jnp.float32(-1e9), w)

            owner = j // _SHARD

            @pl.when(jnp.logical_and(cid == 0, sid == owner))
            def _():
                lane = _iota16()
                row = (jnp.where(lane == 0, bx1, 0.0)
                       + jnp.where(lane == 1, by1, 0.0)
                       + jnp.where(lane == 2, bx2, 0.0)
                       + jnp.where(lane == 3, by2, 0.0)
                       + jnp.where(lane == 4, bsc, 0.0))
                row_buf[...] = row
                pltpu.sync_copy(row_buf, out_h.at[pl.ds(i * 16, 16)])

        @pl.when(jnp.logical_and(jnp.logical_not(valid),
                                 jnp.logical_and(cid == 0, sid == 0)))
        def _():
            row_buf[...] = jnp.zeros((16,), jnp.float32)
            pltpu.sync_copy(row_buf, out_h.at[pl.ds(i * 16, 16)])

        return carry

    lax.fori_loop(0, _POST, round_body, 0)


def _sc_call():
    return functools.partial(
        pl.kernel,
        mesh=plsc.VectorSubcoreMesh(core_axis_name="c",
                                    subcore_axis_name="s"),
        out_type=jax.ShapeDtypeStruct((_OUT_ROWS * 16,), jnp.float32),
        compiler_params=pltpu.CompilerParams(needs_layout_passes=False),
        scratch_types=(
            [pltpu.VMEM((_SHARD,), jnp.float32) for _ in range(8)]
            + [pltpu.VMEM((_SHARD,), jnp.float32) for _ in range(5)]
            + [pltpu.VMEM((_PAD,), jnp.float32) for _ in range(6)]
            + [pltpu.VMEM((_SHARD,), jnp.float32),
               pltpu.VMEM((16,), jnp.int32),
               pltpu.VMEM((_NS * 16,), jnp.int32),
               pltpu.VMEM((16,), jnp.float32)]
            + [pltpu.VMEM_SHARED((_PAD,), jnp.float32)
               for _ in range(5)]
            + [pltpu.VMEM_SHARED((2 * _NS * 16,), jnp.int32)]
        ),
    )


@jax.jit
def kernel(anchors, deltas, scores):
    top_scores, order = jax.lax.top_k(scores, _PRE)
    a = jnp.take(anchors, order, axis=0)
    d = jnp.take(deltas, order, axis=0)
    a_p = jnp.pad(a, ((0, _PAD - _PRE), (0, 0)))
    d_p = jnp.pad(d, ((0, _PAD - _PRE), (0, 0)))
    s_p = jnp.pad(top_scores, (0, _PAD - _PRE), constant_values=-1e30)
    out = _sc_call()(_sc_nms)(
        a_p[:, 0], a_p[:, 1], a_p[:, 2], a_p[:, 3],
        d_p[:, 0], d_p[:, 1], d_p[:, 2], d_p[:, 3], s_p)
    return out.reshape(_OUT_ROWS, 16)[:300, :5]


# speedup vs baseline: 1.4801x; 1.4801x over previous
import functools
import math

import jax
import jax.numpy as jnp
from jax import lax
from jax.experimental import pallas as pl
from jax.experimental.pallas import tpu as pltpu
from jax.experimental.pallas import tpu_sc as plsc

_PRE = 6000
_POST = 300
_THRESH = 0.7
_NS = 16
_PAD = 6144
_SHARD = _PAD // _NS
_BLK = _SHARD // 16
_OUT_ROWS = 304
_BIG = 2 ** 30
_LOG_MAX_RATIO = math.log(1000.0 / 16.0)


def _iota16():
    return lax.broadcasted_iota(jnp.int32, (16,), 0)


def _xlane_min(v):
    dnums = lax.GatherDimensionNumbers(offset_dims=(),
                                       collapsed_slice_dims=(0,),
                                       start_index_map=(0,))
    for s in (8, 4, 2, 1):
        perm = jnp.bitwise_xor(_iota16(), s)
        shuf = lax.gather(v, perm[:, None], dnums, slice_sizes=(1,),
                          mode=lax.GatherScatterMode.PROMISE_IN_BOUNDS)
        v = jnp.minimum(v, shuf)
    return v


def _sc_nms(ax1_h, ay1_h, ax2_h, ay2_h, dx_h, dy_h, dw_h, dh_h, sc_h,
            out_h,
            ia1, ib1, ia2, ib2, idx, idy, idw, idh,
            dx1, dy1, dx2, dy2, dar,
            rep_x1, rep_y1, rep_x2, rep_y2, rep_ar, rep_sc,
            work, mm_buf, cand_buf, acc_vmem, row_buf,
            sh_x1, sh_y1, sh_x2, sh_y2, sh_ar, sh_mm):
    cid = lax.axis_index("c")
    sid = lax.axis_index("s")
    off = sid * _SHARD

    for src, dst in ((ax1_h, ia1), (ay1_h, ib1), (ax2_h, ia2),
                     (ay2_h, ib2), (dx_h, idx), (dy_h, idy),
                     (dw_h, idw), (dh_h, idh)):
        pltpu.sync_copy(src.at[pl.ds(off, _SHARD)], dst)
    pltpu.sync_copy(sc_h.at[pl.ds(off, _SHARD)], work)
    pltpu.sync_copy(sc_h, rep_sc)

    for b in range(_BLK):
        s = b * 16
        a1 = ia1[pl.ds(s, 16)]
        b1 = ib1[pl.ds(s, 16)]
        a2 = ia2[pl.ds(s, 16)]
        b2 = ib2[pl.ds(s, 16)]
        ddx = idx[pl.ds(s, 16)]
        ddy = idy[pl.ds(s, 16)]
        ddw = idw[pl.ds(s, 16)]
        ddh = idh[pl.ds(s, 16)]
        widths = a2 - a1 + 1.0
        heights = b2 - b1 + 1.0
        ctr_x = a1 + 0.5 * widths
        ctr_y = b1 + 0.5 * heights
        ddw = jnp.minimum(ddw, _LOG_MAX_RATIO)
        ddh = jnp.minimum(ddh, _LOG_MAX_RATIO)
        pcx = ddx * widths + ctr_x
        pcy = ddy * heights + ctr_y
        pw = jnp.exp(ddw) * widths
        ph = jnp.exp(ddh) * heights
        x1 = jnp.clip(pcx - 0.5 * pw, 0.0, 1023.0)
        y1 = jnp.clip(pcy - 0.5 * ph, 0.0, 1023.0)
        x2 = jnp.clip(pcx + 0.5 * pw, 0.0, 1023.0)
        y2 = jnp.clip(pcy + 0.5 * ph, 0.0, 1023.0)
        dx1[pl.ds(s, 16)] = x1
        dy1[pl.ds(s, 16)] = y1
        dx2[pl.ds(s, 16)] = x2
        dy2[pl.ds(s, 16)] = y2
        dar[pl.ds(s, 16)] = (x2 - x1 + 1.0) * (y2 - y1 + 1.0)

    for loc, sh in ((dx1, sh_x1), (dy1, sh_y1), (dx2, sh_x2),
                    (dy2, sh_y2), (dar, sh_ar)):
        pltpu.sync_copy(loc, sh.at[pl.ds(off, _SHARD)])
    plsc.subcore_barrier()
    for sh, rep in ((sh_x1, rep_x1), (sh_y1, rep_y1), (sh_x2, rep_x2),
                    (sh_y2, rep_y2), (sh_ar, rep_ar)):
        pltpu.sync_copy(sh, rep)

    lanei = _iota16()

    def round_body(carry):
        it, count, done = carry
        par = lax.rem(it, 2)
        m1 = jnp.full((16,), _BIG, jnp.int32)
        m2 = jnp.full((16,), _BIG, jnp.int32)
        for b in range(_BLK):
            w = work[pl.ds(b * 16, 16)]
            lin = lanei + (off + b * 16)
            c = jnp.where(w > -5e8, lin, _BIG)
            m2 = jnp.minimum(m2, jnp.maximum(m1, c))
            m1 = jnp.minimum(m1, c)
        mm_buf[pl.ds(0, 16)] = m1
        mm_buf[pl.ds(16, 16)] = m2
        pltpu.sync_copy(mm_buf,
                        sh_mm.at[pl.ds(par * (_NS * 32) + sid * 32, 32)])
        plsc.subcore_barrier()
        pltpu.sync_copy(sh_mm.at[pl.ds(par * (_NS * 32), _NS * 32)],
                        cand_buf)
        r1 = jnp.full((16,), _BIG, jnp.int32)
        r2 = jnp.full((16,), _BIG, jnp.int32)
        a2m = jnp.full((16,), _BIG, jnp.int32)
        for t in range(_NS):
            v = cand_buf[pl.ds(t * 32, 16)]
            r2 = jnp.minimum(r2, jnp.maximum(r1, v))
            r1 = jnp.minimum(r1, v)
            a2m = jnp.minimum(a2m, cand_buf[pl.ds(t * 32 + 16, 16)])
        bound = _xlane_min(jnp.minimum(r2, a2m))
        csort, _ = plsc.sort_key_val(r1, r1)
        c0 = csort[0]
        pool = jnp.logical_and(csort < bound, csort < _BIG)

        cg = jnp.minimum(csort, _PAD - 1)
        px1 = plsc.load_gather(rep_x1, [cg])
        py1 = plsc.load_gather(rep_y1, [cg])
        px2 = plsc.load_gather(rep_x2, [cg])
        py2 = plsc.load_gather(rep_y2, [cg])
        pab = plsc.load_gather(rep_ar, [cg])

        acc = pool
        for i in range(16):
            xx1 = jnp.maximum(px1[i], px1)
            yy1 = jnp.maximum(py1[i], py1)
            xx2 = jnp.minimum(px2[i], px2)
            yy2 = jnp.minimum(py2[i], py2)
            iw = jnp.maximum(0.0, xx2 - xx1 + 1.0)
            ih = jnp.maximum(0.0, yy2 - yy1 + 1.0)
            inter = iw * ih
            iou = inter / (pab[i] + pab - inter)
            conflict = jnp.logical_and(
                jnp.logical_and(acc, lanei < i), iou >= _THRESH)
            npc = plsc.all_reduce_population_count(conflict)
            acc = jnp.logical_and(
                acc, jnp.logical_not(
                    jnp.logical_and(lanei == i, npc > 0)))

        rem = _POST - count
        rank = plsc.cumsum(acc.astype(jnp.int32))
        acc = jnp.logical_and(acc, rank <= jnp.broadcast_to(rem, (16,)))
        nacc = plsc.all_reduce_population_count(acc)[0]
        plsc.store_compressed(acc_vmem.at[pl.ds(0, 16)], cg, mask=acc)

        def apply_body(r, _):
            jid = plsc.load_gather(acc_vmem,
                                   [jnp.broadcast_to(r, (16,))])
            bx1 = plsc.load_gather(rep_x1, [jid])
            by1 = plsc.load_gather(rep_y1, [jid])
            bx2 = plsc.load_gather(rep_x2, [jid])
            by2 = plsc.load_gather(rep_y2, [jid])
            bar = plsc.load_gather(rep_ar, [jid])
            for b in range(_BLK):
                s = b * 16
                w = work[pl.ds(s, 16)]
                x1 = dx1[pl.ds(s, 16)]
                y1 = dy1[pl.ds(s, 16)]
                x2 = dx2[pl.ds(s, 16)]
                y2 = dy2[pl.ds(s, 16)]
                ar = dar[pl.ds(s, 16)]
                xx1 = jnp.maximum(bx1, x1)
                yy1 = jnp.maximum(by1, y1)
                xx2 = jnp.minimum(bx2, x2)
                yy2 = jnp.minimum(by2, y2)
                iw = jnp.maximum(0.0, xx2 - xx1 + 1.0)
                ih = jnp.maximum(0.0, yy2 - yy1 + 1.0)
                inter = iw * ih
                iou = inter / (bar + ar - inter)
                lin = lanei + (off + s)
                supp = jnp.logical_or(iou >= _THRESH, lin == jid)
                work[pl.ds(s, 16)] = jnp.where(supp, jnp.float32(-1e9), w)

            owner = jid[0] // _SHARD

            @pl.when(jnp.logical_and(cid == 0, sid == owner))
            def _():
                bsc = plsc.load_gather(rep_sc, [jid])
                row = (jnp.where(lanei == 0, bx1, 0.0)
                       + jnp.where(lanei == 1, by1, 0.0)
                       + jnp.where(lanei == 2, bx2, 0.0)
                       + jnp.where(lanei == 3, by2, 0.0)
                       + jnp.where(lanei == 4, bsc, 0.0))
                row_buf[...] = row
                pltpu.sync_copy(
                    row_buf, out_h.at[pl.ds((count + r) * 16, 16)])
            return 0

        lax.fori_loop(0, nacc, apply_body, 0)
        return (it + 1, count + nacc,
                jnp.logical_or(done, c0 >= _BIG))

    def round_cond(carry):
        _, count, done = carry
        return jnp.logical_and(count < _POST, jnp.logical_not(done))

    _, count_final, _ = lax.while_loop(
        round_cond, round_body,
        (jnp.int32(0), jnp.int32(0), jnp.bool_(False)))

    @pl.when(jnp.logical_and(cid == 0, sid == 0))
    def _():
        def zf(r, _):
            row_buf[...] = jnp.zeros((16,), jnp.float32)
            pltpu.sync_copy(row_buf, out_h.at[pl.ds(r * 16, 16)])
            return 0
        lax.fori_loop(count_final, _POST, zf, 0)


def _sc_call():
    return functools.partial(
        pl.kernel,
        mesh=plsc.VectorSubcoreMesh(core_axis_name="c",
                                    subcore_axis_name="s"),
        out_type=jax.ShapeDtypeStruct((_OUT_ROWS * 16,), jnp.float32),
        compiler_params=pltpu.CompilerParams(needs_layout_passes=False),
        scratch_types=(
            [pltpu.VMEM((_SHARD,), jnp.float32) for _ in range(8)]
            + [pltpu.VMEM((_SHARD,), jnp.float32) for _ in range(5)]
            + [pltpu.VMEM((_PAD,), jnp.float32) for _ in range(6)]
            + [pltpu.VMEM((_SHARD,), jnp.float32),
               pltpu.VMEM((32,), jnp.int32),
               pltpu.VMEM((_NS * 32,), jnp.int32),
               pltpu.VMEM((16,), jnp.int32),
               pltpu.VMEM((16,), jnp.float32)]
            + [pltpu.VMEM_SHARED((_PAD,), jnp.float32)
               for _ in range(5)]
            + [pltpu.VMEM_SHARED((2 * _NS * 32,), jnp.int32)]
        ),
    )


@jax.jit
def kernel(anchors, deltas, scores):
    top_scores, order = jax.lax.top_k(scores, _PRE)
    a = jnp.take(anchors, order, axis=0)
    d = jnp.take(deltas, order, axis=0)
    a_p = jnp.pad(a, ((0, _PAD - _PRE), (0, 0)))
    d_p = jnp.pad(d, ((0, _PAD - _PRE), (0, 0)))
    s_p = jnp.pad(top_scores, (0, _PAD - _PRE), constant_values=-1e30)
    out = _sc_call()(_sc_nms)(
        a_p[:, 0], a_p[:, 1], a_p[:, 2], a_p[:, 3],
        d_p[:, 0], d_p[:, 1], d_p[:, 2], d_p[:, 3], s_p)
    return out.reshape(_OUT_ROWS, 16)[:300, :5]


# speedup vs baseline: 1.5765x; 1.0651x over previous
import functools
import math

import jax
import jax.numpy as jnp
from jax import lax
from jax.experimental import pallas as pl
from jax.experimental.pallas import tpu as pltpu
from jax.experimental.pallas import tpu_sc as plsc

_PRE = 6000
_POST = 300
_THRESH = 0.7
_NS = 16
_PAD = 6144
_SHARD = _PAD // _NS
_BLK = _SHARD // 16
_OUT_ROWS = 304
_BIG = 2 ** 30
_LOG_MAX_RATIO = math.log(1000.0 / 16.0)


def _iota16():
    return lax.broadcasted_iota(jnp.int32, (16,), 0)


def _xlane_min(v):
    dnums = lax.GatherDimensionNumbers(offset_dims=(),
                                       collapsed_slice_dims=(0,),
                                       start_index_map=(0,))
    for s in (8, 4, 2, 1):
        perm = jnp.bitwise_xor(_iota16(), s)
        shuf = lax.gather(v, perm[:, None], dnums, slice_sizes=(1,),
                          mode=lax.GatherScatterMode.PROMISE_IN_BOUNDS)
        v = jnp.minimum(v, shuf)
    return v


def _sc_nms(ax1_h, ay1_h, ax2_h, ay2_h, dx_h, dy_h, dw_h, dh_h, sc_h,
            out_h,
            ia1, ib1, ia2, ib2, idx, idy, idw, idh,
            dx1, dy1, dx2, dy2, dar,
            rep_x1, rep_y1, rep_x2, rep_y2, rep_ar, rep_sc,
            work, mm_buf, cand_buf, acc_vmem, row_buf,
            sh_x1, sh_y1, sh_x2, sh_y2, sh_ar, sh_mm):
    cid = lax.axis_index("c")
    sid = lax.axis_index("s")
    off = sid * _SHARD

    for src, dst in ((ax1_h, ia1), (ay1_h, ib1), (ax2_h, ia2),
                     (ay2_h, ib2), (dx_h, idx), (dy_h, idy),
                     (dw_h, idw), (dh_h, idh)):
        pltpu.sync_copy(src.at[pl.ds(off, _SHARD)], dst)
    pltpu.sync_copy(sc_h.at[pl.ds(off, _SHARD)], work)
    pltpu.sync_copy(sc_h, rep_sc)

    for b in range(_BLK):
        s = b * 16
        a1 = ia1[pl.ds(s, 16)]
        b1 = ib1[pl.ds(s, 16)]
        a2 = ia2[pl.ds(s, 16)]
        b2 = ib2[pl.ds(s, 16)]
        ddx = idx[pl.ds(s, 16)]
        ddy = idy[pl.ds(s, 16)]
        ddw = idw[pl.ds(s, 16)]
        ddh = idh[pl.ds(s, 16)]
        widths = a2 - a1 + 1.0
        heights = b2 - b1 + 1.0
        ctr_x = a1 + 0.5 * widths
        ctr_y = b1 + 0.5 * heights
        ddw = jnp.minimum(ddw, _LOG_MAX_RATIO)
        ddh = jnp.minimum(ddh, _LOG_MAX_RATIO)
        pcx = ddx * widths + ctr_x
        pcy = ddy * heights + ctr_y
        pw = jnp.exp(ddw) * widths
        ph = jnp.exp(ddh) * heights
        x1 = jnp.clip(pcx - 0.5 * pw, 0.0, 1023.0)
        y1 = jnp.clip(pcy - 0.5 * ph, 0.0, 1023.0)
        x2 = jnp.clip(pcx + 0.5 * pw, 0.0, 1023.0)
        y2 = jnp.clip(pcy + 0.5 * ph, 0.0, 1023.0)
        dx1[pl.ds(s, 16)] = x1
        dy1[pl.ds(s, 16)] = y1
        dx2[pl.ds(s, 16)] = x2
        dy2[pl.ds(s, 16)] = y2
        dar[pl.ds(s, 16)] = (x2 - x1 + 1.0) * (y2 - y1 + 1.0)

    for loc, sh in ((dx1, sh_x1), (dy1, sh_y1), (dx2, sh_x2),
                    (dy2, sh_y2), (dar, sh_ar)):
        pltpu.sync_copy(loc, sh.at[pl.ds(off, _SHARD)])
    plsc.subcore_barrier()
    for sh, rep in ((sh_x1, rep_x1), (sh_y1, rep_y1), (sh_x2, rep_x2),
                    (sh_y2, rep_y2), (sh_ar, rep_ar)):
        pltpu.sync_copy(sh, rep)

    lanei = _iota16()

    def round_body(carry):
        it, count, done = carry
        par = lax.rem(it, 2)
        m1 = jnp.full((16,), _BIG, jnp.int32)
        m2 = jnp.full((16,), _BIG, jnp.int32)
        for b in range(_BLK):
            w = work[pl.ds(b * 16, 16)]
            lin = lanei + (off + b * 16)
            c = jnp.where(w > -5e8, lin, _BIG)
            m2 = jnp.minimum(m2, jnp.maximum(m1, c))
            m1 = jnp.minimum(m1, c)
        mm_buf[pl.ds(0, 16)] = m1
        mm_buf[pl.ds(16, 16)] = m2
        pltpu.sync_copy(mm_buf,
                        sh_mm.at[pl.ds(par * (_NS * 32) + sid * 32, 32)])
        plsc.subcore_barrier()
        pltpu.sync_copy(sh_mm.at[pl.ds(par * (_NS * 32), _NS * 32)],
                        cand_buf)
        r1 = jnp.full((16,), _BIG, jnp.int32)
        r2 = jnp.full((16,), _BIG, jnp.int32)
        a2m = jnp.full((16,), _BIG, jnp.int32)
        for t in range(_NS):
            v = cand_buf[pl.ds(t * 32, 16)]
            r2 = jnp.minimum(r2, jnp.maximum(r1, v))
            r1 = jnp.minimum(r1, v)
            a2m = jnp.minimum(a2m, cand_buf[pl.ds(t * 32 + 16, 16)])
        bound = _xlane_min(jnp.minimum(r2, a2m))
        csort, _ = plsc.sort_key_val(r1, r1)
        c0 = csort[0]
        pool = jnp.logical_and(csort < bound, csort < _BIG)

        cg = jnp.minimum(csort, _PAD - 1)
        px1 = plsc.load_gather(rep_x1, [cg])
        py1 = plsc.load_gather(rep_y1, [cg])
        px2 = plsc.load_gather(rep_x2, [cg])
        py2 = plsc.load_gather(rep_y2, [cg])
        pab = plsc.load_gather(rep_ar, [cg])

        acc = pool
        for i in range(16):
            xx1 = jnp.maximum(px1[i], px1)
            yy1 = jnp.maximum(py1[i], py1)
            xx2 = jnp.minimum(px2[i], px2)
            yy2 = jnp.minimum(py2[i], py2)
            iw = jnp.maximum(0.0, xx2 - xx1 + 1.0)
            ih = jnp.maximum(0.0, yy2 - yy1 + 1.0)
            inter = iw * ih
            iou = inter / (pab[i] + pab - inter)
            conflict = jnp.logical_and(
                jnp.logical_and(acc, lanei < i), iou >= _THRESH)
            npc = plsc.all_reduce_population_count(conflict)
            acc = jnp.logical_and(
                acc, jnp.logical_not(
                    jnp.logical_and(lanei == i, npc > 0)))

        rem = _POST - count
        rank = plsc.cumsum(acc.astype(jnp.int32))
        acc = jnp.logical_and(acc, rank <= jnp.broadcast_to(rem, (16,)))
        nacc = plsc.all_reduce_population_count(acc)[0]
        plsc.store_compressed(acc_vmem.at[pl.ds(0, 16)], cg, mask=acc)

        def apply_body(r, _):
            jid = plsc.load_gather(acc_vmem,
                                   [jnp.broadcast_to(r, (16,))])
            bx1 = plsc.load_gather(rep_x1, [jid])
            by1 = plsc.load_gather(rep_y1, [jid])
            bx2 = plsc.load_gather(rep_x2, [jid])
            by2 = plsc.load_gather(rep_y2, [jid])
            bar = plsc.load_gather(rep_ar, [jid])
            for b in range(_BLK):
                s = b * 16
                w = work[pl.ds(s, 16)]
                x1 = dx1[pl.ds(s, 16)]
                y1 = dy1[pl.ds(s, 16)]
                x2 = dx2[pl.ds(s, 16)]
                y2 = dy2[pl.ds(s, 16)]
                ar = dar[pl.ds(s, 16)]
                xx1 = jnp.maximum(bx1, x1)
                yy1 = jnp.maximum(by1, y1)
                xx2 = jnp.minimum(bx2, x2)
                yy2 = jnp.minimum(by2, y2)
                iw = jnp.maximum(0.0, xx2 - xx1 + 1.0)
                ih = jnp.maximum(0.0, yy2 - yy1 + 1.0)
                inter = iw * ih
                iou = inter / (bar + ar - inter)
                lin = lanei + (off + s)
                supp = jnp.logical_or(iou >= _THRESH, lin == jid)
                work[pl.ds(s, 16)] = jnp.where(supp, jnp.float32(-1e9), w)

            owner = jid[0] // _SHARD

            @pl.when(jnp.logical_and(cid == 0, sid == owner))
            def _():
                bsc = plsc.load_gather(rep_sc, [jid])
                row = (jnp.where(lanei == 0, bx1, 0.0)
                       + jnp.where(lanei == 1, by1, 0.0)
                       + jnp.where(lanei == 2, bx2, 0.0)
                       + jnp.where(lanei == 3, by2, 0.0)
                       + jnp.where(lanei == 4, bsc, 0.0))
                row_buf[...] = row
                pltpu.sync_copy(
                    row_buf, out_h.at[pl.ds((count + r) * 16, 16)])
            return 0

        lax.fori_loop(0, nacc, apply_body, 0)
        return (it + 1, count + nacc,
                jnp.logical_or(done, c0 >= _BIG))

    def round_cond(carry):
        _, count, done = carry
        return jnp.logical_and(count < _POST, jnp.logical_not(done))

    _, count_final, _ = lax.while_loop(
        round_cond, round_body,
        (jnp.int32(0), jnp.int32(0), jnp.bool_(False)))

    @pl.when(jnp.logical_and(cid == 0, sid == 0))
    def _():
        def zf(r, _):
            row_buf[...] = jnp.zeros((16,), jnp.float32)
            pltpu.sync_copy(row_buf, out_h.at[pl.ds(r * 16, 16)])
            return 0
        lax.fori_loop(count_final, _POST, zf, 0)


def _sc_call():
    return functools.partial(
        pl.kernel,
        mesh=plsc.VectorSubcoreMesh(core_axis_name="c",
                                    subcore_axis_name="s"),
        out_type=jax.ShapeDtypeStruct((_OUT_ROWS * 16,), jnp.float32),
        compiler_params=pltpu.CompilerParams(needs_layout_passes=False),
        scratch_types=(
            [pltpu.VMEM((_SHARD,), jnp.float32) for _ in range(8)]
            + [pltpu.VMEM((_SHARD,), jnp.float32) for _ in range(5)]
            + [pltpu.VMEM((_PAD,), jnp.float32) for _ in range(6)]
            + [pltpu.VMEM((_SHARD,), jnp.float32),
               pltpu.VMEM((32,), jnp.int32),
               pltpu.VMEM((_NS * 32,), jnp.int32),
               pltpu.VMEM((16,), jnp.int32),
               pltpu.VMEM((16,), jnp.float32)]
            + [pltpu.VMEM_SHARED((_PAD,), jnp.float32)
               for _ in range(5)]
            + [pltpu.VMEM_SHARED((2 * _NS * 32,), jnp.int32)]
        ),
    )


@jax.jit
def kernel(anchors, deltas, scores):
    top_scores, order = jax.lax.top_k(scores, _PRE)
    packed = jnp.concatenate([anchors, deltas], axis=1)
    g = jnp.take(packed, order, axis=0)
    g_p = jnp.pad(g, ((0, _PAD - _PRE), (0, 0)))
    s_p = jnp.pad(top_scores, (0, _PAD - _PRE), constant_values=-1e30)
    out = _sc_call()(_sc_nms)(
        g_p[:, 0], g_p[:, 1], g_p[:, 2], g_p[:, 3],
        g_p[:, 4], g_p[:, 5], g_p[:, 6], g_p[:, 7], s_p)
    return out.reshape(_OUT_ROWS, 16)[:300, :5]


# speedup vs baseline: 1.7150x; 1.0879x over previous
import functools
import math

import jax
import jax.numpy as jnp
from jax import lax
from jax.experimental import pallas as pl
from jax.experimental.pallas import tpu as pltpu
from jax.experimental.pallas import tpu_sc as plsc

_PRE = 6000
_POST = 300
_THRESH = 0.7
_NS = 16
_PAD = 6144
_SHARD = _PAD // _NS
_BLK = _SHARD // 16
_OUT_ROWS = 304
_BIG = 2 ** 30
_LOG_MAX_RATIO = math.log(1000.0 / 16.0)


def _iota16():
    return lax.broadcasted_iota(jnp.int32, (16,), 0)


def _xlane(v, op):
    dnums = lax.GatherDimensionNumbers(offset_dims=(),
                                       collapsed_slice_dims=(0,),
                                       start_index_map=(0,))
    for s in (8, 4, 2, 1):
        perm = jnp.bitwise_xor(_iota16(), s)
        shuf = lax.gather(v, perm[:, None], dnums, slice_sizes=(1,),
                          mode=lax.GatherScatterMode.PROMISE_IN_BOUNDS)
        v = op(v, shuf)
    return v


def _xlane_min(v):
    return _xlane(v, jnp.minimum)


def _xlane_any(v):
    return _xlane(v.astype(jnp.int32), jnp.bitwise_or) > 0


def _sc_nms(ax1_h, ay1_h, ax2_h, ay2_h, dx_h, dy_h, dw_h, dh_h, sc_h,
            out_h,
            ia1, ib1, ia2, ib2, idx, idy, idw, idh,
            dx1, dy1, dx2, dy2, dar,
            rep_x1, rep_y1, rep_x2, rep_y2, rep_ar, rep_sc,
            work, mm_buf, cand_buf, acc_vmem, staging, dummy, dsem,
            sh_x1, sh_y1, sh_x2, sh_y2, sh_ar, sh_mm):
    cid = lax.axis_index("c")
    sid = lax.axis_index("s")
    off = sid * _SHARD

    for src, dst in ((ax1_h, ia1), (ay1_h, ib1), (ax2_h, ia2),
                     (ay2_h, ib2), (dx_h, idx), (dy_h, idy),
                     (dw_h, idw), (dh_h, idh)):
        pltpu.sync_copy(src.at[pl.ds(off, _SHARD)], dst)
    pltpu.sync_copy(sc_h.at[pl.ds(off, _SHARD)], work)
    pltpu.sync_copy(sc_h, rep_sc)

    for b in range(_BLK):
        s = b * 16
        a1 = ia1[pl.ds(s, 16)]
        b1 = ib1[pl.ds(s, 16)]
        a2 = ia2[pl.ds(s, 16)]
        b2 = ib2[pl.ds(s, 16)]
        ddx = idx[pl.ds(s, 16)]
        ddy = idy[pl.ds(s, 16)]
        ddw = idw[pl.ds(s, 16)]
        ddh = idh[pl.ds(s, 16)]
        widths = a2 - a1 + 1.0
        heights = b2 - b1 + 1.0
        ctr_x = a1 + 0.5 * widths
        ctr_y = b1 + 0.5 * heights
        ddw = jnp.minimum(ddw, _LOG_MAX_RATIO)
        ddh = jnp.minimum(ddh, _LOG_MAX_RATIO)
        pcx = ddx * widths + ctr_x
        pcy = ddy * heights + ctr_y
        pw = jnp.exp(ddw) * widths
        ph = jnp.exp(ddh) * heights
        x1 = jnp.clip(pcx - 0.5 * pw, 0.0, 1023.0)
        y1 = jnp.clip(pcy - 0.5 * ph, 0.0, 1023.0)
        x2 = jnp.clip(pcx + 0.5 * pw, 0.0, 1023.0)
        y2 = jnp.clip(pcy + 0.5 * ph, 0.0, 1023.0)
        dx1[pl.ds(s, 16)] = x1
        dy1[pl.ds(s, 16)] = y1
        dx2[pl.ds(s, 16)] = x2
        dy2[pl.ds(s, 16)] = y2
        dar[pl.ds(s, 16)] = (x2 - x1 + 1.0) * (y2 - y1 + 1.0)

    for loc, sh in ((dx1, sh_x1), (dy1, sh_y1), (dx2, sh_x2),
                    (dy2, sh_y2), (dar, sh_ar)):
        pltpu.sync_copy(loc, sh.at[pl.ds(off, _SHARD)])
    plsc.subcore_barrier()
    for sh, rep in ((sh_x1, rep_x1), (sh_y1, rep_y1), (sh_x2, rep_x2),
                    (sh_y2, rep_y2), (sh_ar, rep_ar)):
        pltpu.sync_copy(sh, rep)

    lanei = _iota16()

    def round_body(carry):
        it, count, done, wc = carry
        par = lax.rem(it, 2)
        m1 = jnp.full((16,), _BIG, jnp.int32)
        m2 = jnp.full((16,), _BIG, jnp.int32)
        for b in range(_BLK):
            w = work[pl.ds(b * 16, 16)]
            lin = lanei + (off + b * 16)
            c = jnp.where(w > -5e8, lin, _BIG)
            m2 = jnp.minimum(m2, jnp.maximum(m1, c))
            m1 = jnp.minimum(m1, c)
        mm_buf[pl.ds(0, 16)] = m1
        mm_buf[pl.ds(16, 16)] = m2
        pltpu.sync_copy(mm_buf,
                        sh_mm.at[pl.ds(par * (_NS * 32) + sid * 32, 32)])
        plsc.subcore_barrier()
        pltpu.sync_copy(sh_mm.at[pl.ds(par * (_NS * 32), _NS * 32)],
                        cand_buf)
        r1 = jnp.full((16,), _BIG, jnp.int32)
        r2 = jnp.full((16,), _BIG, jnp.int32)
        a2m = jnp.full((16,), _BIG, jnp.int32)
        for t in range(_NS):
            v = cand_buf[pl.ds(t * 32, 16)]
            r2 = jnp.minimum(r2, jnp.maximum(r1, v))
            r1 = jnp.minimum(r1, v)
            a2m = jnp.minimum(a2m, cand_buf[pl.ds(t * 32 + 16, 16)])
        bound = _xlane_min(jnp.minimum(r2, a2m))
        csort, _ = plsc.sort_key_val(r1, r1)
        c0 = csort[0]
        pool = jnp.logical_and(csort < bound, csort < _BIG)

        cg = jnp.minimum(csort, _PAD - 1)
        px1 = plsc.load_gather(rep_x1, [cg])
        py1 = plsc.load_gather(rep_y1, [cg])
        px2 = plsc.load_gather(rep_x2, [cg])
        py2 = plsc.load_gather(rep_y2, [cg])
        pab = plsc.load_gather(rep_ar, [cg])

        acc = pool
        for i in range(16):
            xx1 = jnp.maximum(px1[i], px1)
            yy1 = jnp.maximum(py1[i], py1)
            xx2 = jnp.minimum(px2[i], px2)
            yy2 = jnp.minimum(py2[i], py2)
            iw = jnp.maximum(0.0, xx2 - xx1 + 1.0)
            ih = jnp.maximum(0.0, yy2 - yy1 + 1.0)
            inter = iw * ih
            iou = inter / (pab[i] + pab - inter)
            conflict = jnp.logical_and(
                jnp.logical_and(acc, lanei < i), iou >= _THRESH)
            hit = _xlane_any(conflict)
            acc = jnp.logical_and(
                acc, jnp.logical_not(
                    jnp.logical_and(lanei == i, hit)))

        rem = _POST - count
        rank = plsc.cumsum(acc.astype(jnp.int32))
        acc = jnp.logical_and(acc, rank <= jnp.broadcast_to(rem, (16,)))
        nacc = plsc.all_reduce_population_count(acc)[0]
        plsc.store_compressed(acc_vmem.at[pl.ds(0, 16)], cg, mask=acc)

        def apply_body(r, awc):
            jid = plsc.load_gather(acc_vmem,
                                   [jnp.broadcast_to(r, (16,))])
            bx1 = plsc.load_gather(rep_x1, [jid])
            by1 = plsc.load_gather(rep_y1, [jid])
            bx2 = plsc.load_gather(rep_x2, [jid])
            by2 = plsc.load_gather(rep_y2, [jid])
            bar = plsc.load_gather(rep_ar, [jid])
            for b in range(_BLK):
                s = b * 16
                w = work[pl.ds(s, 16)]
                x1 = dx1[pl.ds(s, 16)]
                y1 = dy1[pl.ds(s, 16)]
                x2 = dx2[pl.ds(s, 16)]
                y2 = dy2[pl.ds(s, 16)]
                ar = dar[pl.ds(s, 16)]
                xx1 = jnp.maximum(bx1, x1)
                yy1 = jnp.maximum(by1, y1)
                xx2 = jnp.minimum(bx2, x2)
                yy2 = jnp.minimum(by2, y2)
                iw = jnp.maximum(0.0, xx2 - xx1 + 1.0)
                ih = jnp.maximum(0.0, yy2 - yy1 + 1.0)
                inter = iw * ih
                iou = inter / (bar + ar - inter)
                lin = lanei + (off + s)
                supp = jnp.logical_or(iou >= _THRESH, lin == jid)
                work[pl.ds(s, 16)] = jnp.where(supp, jnp.float32(-1e9), w)

            owner = jid[0] // _SHARD
            is_writer = jnp.logical_and(cid == 0, sid == owner)

            @pl.when(is_writer)
            def _():
                bsc = plsc.load_gather(rep_sc, [jid])
                row = (jnp.where(lanei == 0, bx1, 0.0)
                       + jnp.where(lanei == 1, by1, 0.0)
                       + jnp.where(lanei == 2, bx2, 0.0)
                       + jnp.where(lanei == 3, by2, 0.0)
                       + jnp.where(lanei == 4, bsc, 0.0))
                o = (count + r) * 16
                staging[pl.ds(o, 16)] = row
                pltpu.async_copy(staging.at[pl.ds(o, 16)],
                                 out_h.at[pl.ds(o, 16)], dsem)
            return awc + is_writer.astype(jnp.int32)

        wc = lax.fori_loop(0, nacc, apply_body, wc)
        return (it + 1, count + nacc,
                jnp.logical_or(done, c0 >= _BIG), wc)

    def round_cond(carry):
        _, count, done, _ = carry
        return jnp.logical_and(count < _POST, jnp.logical_not(done))

    _, count_final, _, wc_end = lax.while_loop(
        round_cond, round_body,
        (jnp.int32(0), jnp.int32(0), jnp.bool_(False), jnp.int32(0)))

    tile00 = jnp.logical_and(cid == 0, sid == 0)

    @pl.when(tile00)
    def _():
        def zf(r, _):
            staging[pl.ds(r * 16, 16)] = jnp.zeros((16,), jnp.float32)
            pltpu.async_copy(staging.at[pl.ds(r * 16, 16)],
                             out_h.at[pl.ds(r * 16, 16)], dsem)
            return 0
        lax.fori_loop(count_final, _POST, zf, 0)

    wc_total = wc_end + jnp.where(tile00, _POST - count_final,
                                  jnp.int32(0))

    def drain(r, _):
        pltpu.make_async_copy(out_h.at[pl.ds(_POST * 16, 16)],
                              dummy, dsem).wait()
        return 0
    lax.fori_loop(0, wc_total, drain, 0)


def _sc_call():
    return functools.partial(
        pl.kernel,
        mesh=plsc.VectorSubcoreMesh(core_axis_name="c",
                                    subcore_axis_name="s"),
        out_type=jax.ShapeDtypeStruct((_OUT_ROWS * 16,), jnp.float32),
        compiler_params=pltpu.CompilerParams(needs_layout_passes=False),
        scratch_types=(
            [pltpu.VMEM((_SHARD,), jnp.float32) for _ in range(8)]
            + [pltpu.VMEM((_SHARD,), jnp.float32) for _ in range(5)]
            + [pltpu.VMEM((_PAD,), jnp.float32) for _ in range(6)]
            + [pltpu.VMEM((_SHARD,), jnp.float32),
               pltpu.VMEM((32,), jnp.int32),
               pltpu.VMEM((_NS * 32,), jnp.int32),
               pltpu.VMEM((16,), jnp.int32),
               pltpu.VMEM((_OUT_ROWS * 16,), jnp.float32),
               pltpu.VMEM((16,), jnp.float32),
               pltpu.SemaphoreType.DMA]
            + [pltpu.VMEM_SHARED((_PAD,), jnp.float32)
               for _ in range(5)]
            + [pltpu.VMEM_SHARED((2 * _NS * 32,), jnp.int32)]
        ),
    )


@jax.jit
def kernel(anchors, deltas, scores):
    top_scores, order = jax.lax.top_k(scores, _PRE)
    packed = jnp.concatenate([anchors, deltas], axis=1)
    g = jnp.take(packed, order, axis=0)
    g_p = jnp.pad(g, ((0, _PAD - _PRE), (0, 0)))
    s_p = jnp.pad(top_scores, (0, _PAD - _PRE), constant_values=-1e30)
    out = _sc_call()(_sc_nms)(
        g_p[:, 0], g_p[:, 1], g_p[:, 2], g_p[:, 3],
        g_p[:, 4], g_p[:, 5], g_p[:, 6], g_p[:, 7], s_p)
    return out.reshape(_OUT_ROWS, 16)[:300, :5]
